# baseline (speedup 1.0000x reference)
import functools

import jax
import jax.numpy as jnp
from jax.experimental import pallas as pl
from jax.experimental.pallas import tpu as pltpu

D_MODEL = 1024
D_FF = 2048
N_EXPERTS = 8
MAX_K = 4
THRESHOLD = 0.8
T_TOKENS = 2048

T_BLK = 256
F_BLK = 1024

_HI = jax.lax.Precision.HIGHEST


def _moe_kernel(x_ref, logits_ref, wg_ref, wu_ref, wd_ref, out_ref, gates_ref):
    e = pl.program_id(0)
    f = pl.program_id(1)
    t = pl.program_id(2)

    x = x_ref[...]

    @pl.when((e == 0) & (f == 0))
    def _():
        logits = logits_ref[...]
        m = jnp.max(logits, axis=-1, keepdims=True)
        ex = jnp.exp(logits - m)
        p = ex / jnp.sum(ex, axis=-1, keepdims=True)
        lane = jax.lax.broadcasted_iota(jnp.int32, p.shape, 1)
        cols = []
        for j in range(N_EXPERTS):
            pj = p[:, j:j + 1]
            above = (p > pj) | ((p == pj) & (lane < j))
            rank_j = jnp.sum(above.astype(jnp.float32), axis=-1, keepdims=True)
            csum_before = jnp.sum(jnp.where(above, p, 0.0), axis=-1,
                                  keepdims=True)
            mask = (csum_before < THRESHOLD) & (rank_j < MAX_K)
            cols.append(jnp.where(mask, pj, 0.0))
        gates_ref[pl.ds(t * T_BLK, T_BLK), :] = jnp.concatenate(cols, axis=-1)

    wg = wg_ref[0]
    wu = wu_ref[0]
    wd = wd_ref[0]

    hg = jnp.dot(x, wg, preferred_element_type=jnp.float32, precision=_HI)
    hu = jnp.dot(x, wu, preferred_element_type=jnp.float32, precision=_HI)
    h = hg * jax.nn.sigmoid(hg) * hu
    y = jnp.dot(h, wd, preferred_element_type=jnp.float32, precision=_HI)

    gates = gates_ref[pl.ds(t * T_BLK, T_BLK), :]
    lane = jax.lax.broadcasted_iota(jnp.int32, gates.shape, 1)
    g = jnp.sum(jnp.where(lane == e, gates, 0.0), axis=-1, keepdims=True)

    contrib = y * g

    @pl.when((e == 0) & (f == 0))
    def _():
        out_ref[pl.ds(t * T_BLK, T_BLK), :] = contrib

    @pl.when((e != 0) | (f != 0))
    def _():
        out_ref[pl.ds(t * T_BLK, T_BLK), :] = (
            out_ref[pl.ds(t * T_BLK, T_BLK), :] + contrib)


@jax.jit
def kernel(x, W_router, W_gate, W_up, W_down):
    logits = x @ W_router
    grid = (N_EXPERTS, D_FF // F_BLK, T_TOKENS // T_BLK)
    out = pl.pallas_call(
        _moe_kernel,
        grid=grid,
        in_specs=[
            pl.BlockSpec((T_BLK, D_MODEL), lambda e, f, t: (t, 0)),
            pl.BlockSpec((T_BLK, N_EXPERTS), lambda e, f, t: (t, 0)),
            pl.BlockSpec((1, D_MODEL, F_BLK), lambda e, f, t: (e, 0, f)),
            pl.BlockSpec((1, D_MODEL, F_BLK), lambda e, f, t: (e, 0, f)),
            pl.BlockSpec((1, F_BLK, D_MODEL), lambda e, f, t: (e, f, 0)),
        ],
        out_specs=pl.BlockSpec((T_TOKENS, D_MODEL), lambda e, f, t: (0, 0)),
        out_shape=jax.ShapeDtypeStruct((T_TOKENS, D_MODEL), jnp.float32),
        scratch_shapes=[pltpu.VMEM((T_TOKENS, N_EXPERTS), jnp.float32)],
        compiler_params=pltpu.CompilerParams(
            dimension_semantics=("arbitrary", "arbitrary", "arbitrary"),
        ),
    )(x, logits, W_gate, W_up, W_down)
    return out


# speedup vs baseline: 3.3954x; 3.3954x over previous
import functools

import jax
import jax.numpy as jnp
from jax.experimental import pallas as pl
from jax.experimental.pallas import tpu as pltpu

D_MODEL = 1024
D_FF = 2048
N_EXPERTS = 8
MAX_K = 4
THRESHOLD = 0.8
T_TOKENS = 2048

T_BLK = 256
F_BLK = 1024

_HI = jax.lax.Precision.HIGHEST


def _moe_kernel(x_ref, logits_ref, wg_ref, wu_ref, wd_ref, out_ref, gates_ref):
    e = pl.program_id(0)
    f = pl.program_id(1)
    t = pl.program_id(2)

    x = x_ref[...]

    @pl.when((e == 0) & (f == 0))
    def _():
        logits = logits_ref[...]
        m = jnp.max(logits, axis=-1, keepdims=True)
        ex = jnp.exp(logits - m)
        p = ex / jnp.sum(ex, axis=-1, keepdims=True)
        lane = jax.lax.broadcasted_iota(jnp.int32, p.shape, 1)
        cols = []
        for j in range(N_EXPERTS):
            pj = p[:, j:j + 1]
            above = (p > pj) | ((p == pj) & (lane < j))
            rank_j = jnp.sum(above.astype(jnp.float32), axis=-1, keepdims=True)
            csum_before = jnp.sum(jnp.where(above, p, 0.0), axis=-1,
                                  keepdims=True)
            mask = (csum_before < THRESHOLD) & (rank_j < MAX_K)
            cols.append(jnp.where(mask, pj, 0.0))
        gates_ref[pl.ds(t * T_BLK, T_BLK), :] = jnp.concatenate(cols, axis=-1)

    wg = wg_ref[0]
    wu = wu_ref[0]
    wd = wd_ref[0]

    hg = jnp.dot(x, wg, preferred_element_type=jnp.float32)
    hu = jnp.dot(x, wu, preferred_element_type=jnp.float32)
    h = (hg * jax.nn.sigmoid(hg) * hu).astype(jnp.bfloat16)
    y = jnp.dot(h, wd, preferred_element_type=jnp.float32)

    gates = gates_ref[pl.ds(t * T_BLK, T_BLK), :]
    lane = jax.lax.broadcasted_iota(jnp.int32, gates.shape, 1)
    g = jnp.sum(jnp.where(lane == e, gates, 0.0), axis=-1, keepdims=True)

    contrib = y * g

    @pl.when((e == 0) & (f == 0))
    def _():
        out_ref[pl.ds(t * T_BLK, T_BLK), :] = contrib

    @pl.when((e != 0) | (f != 0))
    def _():
        out_ref[pl.ds(t * T_BLK, T_BLK), :] = (
            out_ref[pl.ds(t * T_BLK, T_BLK), :] + contrib)


@jax.jit
def kernel(x, W_router, W_gate, W_up, W_down):
    logits = x @ W_router
    xb = x.astype(jnp.bfloat16)
    wg = W_gate.astype(jnp.bfloat16)
    wu = W_up.astype(jnp.bfloat16)
    wd = W_down.astype(jnp.bfloat16)
    grid = (N_EXPERTS, D_FF // F_BLK, T_TOKENS // T_BLK)
    out = pl.pallas_call(
        _moe_kernel,
        grid=grid,
        in_specs=[
            pl.BlockSpec((T_BLK, D_MODEL), lambda e, f, t: (t, 0)),
            pl.BlockSpec((T_BLK, N_EXPERTS), lambda e, f, t: (t, 0)),
            pl.BlockSpec((1, D_MODEL, F_BLK), lambda e, f, t: (e, 0, f)),
            pl.BlockSpec((1, D_MODEL, F_BLK), lambda e, f, t: (e, 0, f)),
            pl.BlockSpec((1, F_BLK, D_MODEL), lambda e, f, t: (e, f, 0)),
        ],
        out_specs=pl.BlockSpec((T_TOKENS, D_MODEL), lambda e, f, t: (0, 0)),
        out_shape=jax.ShapeDtypeStruct((T_TOKENS, D_MODEL), jnp.float32),
        scratch_shapes=[pltpu.VMEM((T_TOKENS, N_EXPERTS), jnp.float32)],
        compiler_params=pltpu.CompilerParams(
            dimension_semantics=("arbitrary", "arbitrary", "arbitrary"),
        ),
    )(xb, logits, wg, wu, wd)
    return out
